# baseline (speedup 1.0000x reference)
import functools

import jax
import jax.numpy as jnp
from jax import lax
from jax.experimental import pallas as pl
from jax.experimental.pallas import tpu as pltpu
from jax.experimental.pallas import tpu_sc as plsc

_USER_NUM = 500000
_LANES = 16


def _make_sc_kernel(B, D, nw):
    b_per_w = B // nw
    n_chunks = b_per_w // 128
    n_groups = b_per_w // _LANES

    mesh = plsc.VectorSubcoreMesh(core_axis_name="c", subcore_axis_name="s")

    @functools.partial(
        pl.kernel,
        mesh=mesh,
        compiler_params=pltpu.CompilerParams(
            needs_layout_passes=False, use_tc_tiling_on_sc=False),
        out_type=jax.ShapeDtypeStruct((B,), jnp.float32),
        scratch_types=[
            pltpu.VMEM((n_chunks, 128), jnp.int32),
            pltpu.VMEM((n_chunks, 128), jnp.int32),
            pltpu.VMEM((b_per_w, D), jnp.float32),
            pltpu.VMEM((b_per_w, D), jnp.float32),
            pltpu.VMEM((b_per_w,), jnp.float32),
            pltpu.SemaphoreType.DMA,
        ],
    )
    def sc_kernel(emb_hbm, u_hbm, v_hbm, out_hbm, idx_u, idx_v, rows_u,
                  rows_v, out_v, sem):
        wid = lax.axis_index("s") * 2 + lax.axis_index("c")
        row0 = wid * n_chunks

        pltpu.sync_copy(u_hbm.at[pl.ds(row0, n_chunks)], idx_u)
        pltpu.sync_copy(v_hbm.at[pl.ds(row0, n_chunks)], idx_v)

        descs = []
        for j in range(n_chunks):
            dst = pl.ds(j * 128, 128)
            descs.append(pltpu.async_copy(emb_hbm.at[idx_u.at[j]],
                                          rows_u.at[dst], sem))
            descs.append(pltpu.async_copy(emb_hbm.at[idx_v.at[j]],
                                          rows_v.at[dst], sem))
        for d in descs:
            d.wait()

        iota = lax.iota(jnp.int32, _LANES)

        def group(g, carry):
            row = g * _LANES + iota
            acc = jnp.zeros((_LANES,), jnp.float32)
            for d in range(D):
                col = jnp.full((_LANES,), d, jnp.int32)
                gu = plsc.load_gather(rows_u, [row, col])
                gv = plsc.load_gather(rows_v, [row, col])
                acc = acc + gu * gv
            out_v[pl.ds(g * _LANES, _LANES)] = acc
            return carry

        lax.fori_loop(0, n_groups, group, 0)

        pltpu.sync_copy(out_v, out_hbm.at[pl.ds(wid * b_per_w, b_per_w)])

    return sc_kernel


def kernel(u, v, embedding):
    B = u.shape[0]
    D = embedding.shape[1]
    info = plsc.get_sparse_core_info()
    nw = info.num_cores * info.num_subcores

    u2 = u.astype(jnp.int32).reshape(B // 128, 128)
    v2 = (v.astype(jnp.int32) + _USER_NUM).reshape(B // 128, 128)

    sc = _make_sc_kernel(B, D, nw)
    return sc(embedding, u2, v2)
